# baseline (speedup 1.0000x reference)
import functools

import jax
import jax.numpy as jnp
from jax import lax
from jax.experimental import pallas as pl
from jax.experimental.pallas import tpu as pltpu
from jax.experimental.pallas import tpu_sc as plsc

N_NODES = 10000
N_FEAT = 128
N_EDGES = 320000
T_PAD = 10240

NC = 2
NS = 16
NW = NC * NS
LANES = 16

SLAB = 10240
PIECE_SIZES = (1280, 2560, 3200, 3200)
LAST_BASE = N_EDGES - SLAB
TAIL = N_EDGES - (NW - 1) * SLAB

CLAMP_MIN = 1e-05
CLAMP_MAX = 0.99999

ROW_BLOCK = 2048
N_ROW_BLOCKS = -(-N_NODES // ROW_BLOCK)


def _tc_scores_body(x_ref, w_ref, s_src_ref, s_dst_ref):
    res = lax.dot_general(
        w_ref[...],
        x_ref[...],
        (((1,), (1,)), ((), ())),
        preferred_element_type=jnp.float32,
        precision=lax.Precision.DEFAULT,
    )
    s_src_ref[...] = res[0:1, :]
    s_dst_ref[...] = res[1:2, :]


def _sc_edge_body(
    edge_ref, s1_ref, s2_ref, out_ref, ev, out_v, t_src, t_dst, sem, osem
):
    wid = lax.axis_index("s") * NC + lax.axis_index("c")
    is_last = wid == NW - 1
    base = jnp.where(is_last, LAST_BASE, wid * SLAB)

    c1 = pltpu.async_copy(s1_ref.at[0], t_src, sem)
    c2 = pltpu.async_copy(s2_ref.at[0], t_dst, sem)
    offs = [sum(PIECE_SIZES[:p]) for p in range(len(PIECE_SIZES))]
    pieces = [
        pltpu.async_copy(
            edge_ref.at[:, pl.ds(base + off, sz)],
            ev.at[:, off : off + sz],
            sem,
        )
        for off, sz in zip(offs, PIECE_SIZES)
    ]
    c1.wait()
    c2.wait()

    def make_step(lo, size):
        @plsc.parallel_loop(lo, lo + size, LANES, unroll=8)
        def _step(i):
            a = plsc.load_gather(t_src, [ev[0, pl.ds(i, LANES)]]) + plsc.load_gather(
                t_dst, [ev[1, pl.ds(i, LANES)]]
            )
            sig = 1.0 / (1.0 + jnp.exp(-a))
            out_v[pl.ds(i, LANES)] = jnp.clip(sig, CLAMP_MIN, CLAMP_MAX)

    wbs = []
    for p, (off, sz) in enumerate(zip(offs, PIECE_SIZES)):
        pieces[p].wait()
        make_step(off, sz)
        wbs.append(
            pltpu.async_copy(
                out_v.at[pl.ds(off, sz)],
                out_ref.at[pl.ds(base + off, sz)],
                osem,
            )
        )

    for wb in wbs:
        wb.wait()


@jax.jit
def kernel(x, edge_index, att_weight):
    w = att_weight.reshape(2, N_FEAT)
    s_src, s_dst = pl.pallas_call(
        _tc_scores_body,
        grid=(N_ROW_BLOCKS,),
        in_specs=[
            pl.BlockSpec((ROW_BLOCK, N_FEAT), lambda i: (i, 0)),
            pl.BlockSpec((2, N_FEAT), lambda i: (0, 0)),
        ],
        out_specs=[
            pl.BlockSpec((1, ROW_BLOCK), lambda i: (0, i)),
            pl.BlockSpec((1, ROW_BLOCK), lambda i: (0, i)),
        ],
        out_shape=[
            jax.ShapeDtypeStruct((1, T_PAD), jnp.float32),
            jax.ShapeDtypeStruct((1, T_PAD), jnp.float32),
        ],
    )(x, w)

    edges = edge_index.astype(jnp.int32)
    sc_call = pl.kernel(
        _sc_edge_body,
        out_type=jax.ShapeDtypeStruct((N_EDGES,), jnp.float32),
        mesh=plsc.VectorSubcoreMesh(
            core_axis_name="c", subcore_axis_name="s", num_cores=NC, num_subcores=NS
        ),
        compiler_params=pltpu.CompilerParams(needs_layout_passes=False),
        scratch_types=[
            pltpu.VMEM((2, SLAB), jnp.int32),
            pltpu.VMEM((SLAB,), jnp.float32),
            pltpu.VMEM((T_PAD,), jnp.float32),
            pltpu.VMEM((T_PAD,), jnp.float32),
            pltpu.SemaphoreType.DMA,
            pltpu.SemaphoreType.DMA,
        ],
    )
    return sc_call(edges, s_src, s_dst)


# speedup vs baseline: 1.0380x; 1.0380x over previous
import functools

import jax
import jax.numpy as jnp
from jax import lax
from jax.experimental import pallas as pl
from jax.experimental.pallas import tpu as pltpu
from jax.experimental.pallas import tpu_sc as plsc

N_NODES = 10000
N_FEAT = 128
N_EDGES = 320000
T_PAD = 10240

NC = 2
NS = 16
NW = NC * NS
LANES = 16

SLAB = 10240
PIECE_SIZES = (1280, 2560, 3200, 3200)
LAST_BASE = N_EDGES - SLAB
TAIL = N_EDGES - (NW - 1) * SLAB

CLAMP_MIN = 1e-05
CLAMP_MAX = 0.99999

ROW_BLOCK = 8192
N_ROW_BLOCKS = -(-N_NODES // ROW_BLOCK)


def _tc_scores_body(x_ref, w_ref, s_src_ref, s_dst_ref):
    res = lax.dot_general(
        w_ref[...],
        x_ref[...],
        (((1,), (1,)), ((), ())),
        preferred_element_type=jnp.float32,
        precision=lax.Precision.DEFAULT,
    )
    s_src_ref[...] = res[0:1, :]
    s_dst_ref[...] = res[1:2, :]


def _sc_edge_body(
    edge_ref, s1_ref, s2_ref, out_ref, ev, out_v, t_src, t_dst, sem, osem
):
    wid = lax.axis_index("s") * NC + lax.axis_index("c")
    is_last = wid == NW - 1
    base = jnp.where(is_last, LAST_BASE, wid * SLAB)

    c1 = pltpu.async_copy(s1_ref.at[0], t_src, sem)
    c2 = pltpu.async_copy(s2_ref.at[0], t_dst, sem)
    offs = [sum(PIECE_SIZES[:p]) for p in range(len(PIECE_SIZES))]
    pieces = [
        pltpu.async_copy(
            edge_ref.at[:, pl.ds(base + off, sz)],
            ev.at[:, off : off + sz],
            sem,
        )
        for off, sz in zip(offs, PIECE_SIZES)
    ]
    c1.wait()
    c2.wait()

    def make_step(lo, size):
        @plsc.parallel_loop(lo, lo + size, LANES, unroll=8)
        def _step(i):
            a = plsc.load_gather(t_src, [ev[0, pl.ds(i, LANES)]]) + plsc.load_gather(
                t_dst, [ev[1, pl.ds(i, LANES)]]
            )
            sig = 1.0 / (1.0 + jnp.exp(-a))
            out_v[pl.ds(i, LANES)] = jnp.clip(sig, CLAMP_MIN, CLAMP_MAX)

    wbs = []
    for p, (off, sz) in enumerate(zip(offs, PIECE_SIZES)):
        pieces[p].wait()
        make_step(off, sz)
        wbs.append(
            pltpu.async_copy(
                out_v.at[pl.ds(off, sz)],
                out_ref.at[pl.ds(base + off, sz)],
                osem,
            )
        )

    for wb in wbs:
        wb.wait()


@jax.jit
def kernel(x, edge_index, att_weight):
    w = att_weight.reshape(2, N_FEAT)
    s_src, s_dst = pl.pallas_call(
        _tc_scores_body,
        grid=(N_ROW_BLOCKS,),
        in_specs=[
            pl.BlockSpec((ROW_BLOCK, N_FEAT), lambda i: (i, 0)),
            pl.BlockSpec((2, N_FEAT), lambda i: (0, 0)),
        ],
        out_specs=[
            pl.BlockSpec((1, ROW_BLOCK), lambda i: (0, i)),
            pl.BlockSpec((1, ROW_BLOCK), lambda i: (0, i)),
        ],
        out_shape=[
            jax.ShapeDtypeStruct((1, T_PAD), jnp.float32),
            jax.ShapeDtypeStruct((1, T_PAD), jnp.float32),
        ],
    )(x, w)

    edges = edge_index.astype(jnp.int32)
    sc_call = pl.kernel(
        _sc_edge_body,
        out_type=jax.ShapeDtypeStruct((N_EDGES,), jnp.float32),
        mesh=plsc.VectorSubcoreMesh(
            core_axis_name="c", subcore_axis_name="s", num_cores=NC, num_subcores=NS
        ),
        compiler_params=pltpu.CompilerParams(needs_layout_passes=False),
        scratch_types=[
            pltpu.VMEM((2, SLAB), jnp.int32),
            pltpu.VMEM((SLAB,), jnp.float32),
            pltpu.VMEM((T_PAD,), jnp.float32),
            pltpu.VMEM((T_PAD,), jnp.float32),
            pltpu.SemaphoreType.DMA,
            pltpu.SemaphoreType.DMA,
        ],
    )
    return sc_call(edges, s_src, s_dst)
